# Initial kernel scaffold; baseline (speedup 1.0000x reference)
import jax
import jax.numpy as jnp
from jax.experimental import pallas as pl


def kernel(features, adjacency, mask, W1, b1, W2, b2, Wg, bg, Wd, bd, Wp1, bp1, Wp2, bp2, Wpi, bpi):
    raise NotImplementedError("write your pallas kernel here")


# speedup vs baseline: 1477.6861x; 1477.6861x over previous
import jax
import jax.numpy as jnp
from jax.experimental import pallas as pl

N = 1000


def _body(feat_ref, adj_ref, mask_ref, W1_ref, b1_ref, W2_ref, b2_ref,
          Wg_ref, bg_ref, Wd_ref, bd_ref, Wp1a_ref, Wp1b_ref, bp1_ref,
          Wp2_ref, bp2_ref, Wpi_ref, bpi_ref, out_ref):
    f32 = jnp.float32

    def mm(a, b):
        return jax.lax.dot_general(a, b, (((1,), (0,)), ((), ())),
                                   preferred_element_type=f32)

    def mmT(a, b):
        return jax.lax.dot_general(a, b, (((0,), (0,)), ((), ())),
                                   preferred_element_type=f32)

    X = jnp.maximum(mm(feat_ref[...], W1_ref[...]) + b1_ref[...], 0.0)
    X = jnp.maximum(mm(X, W2_ref[...]) + b2_ref[...], 0.0)

    A = adj_ref[...]
    deg_r = jnp.sum(A, axis=0, keepdims=True)
    dinv_r = jnp.where(deg_r > 0, jax.lax.rsqrt(deg_r), 0.0)
    ones_c = jnp.ones((N, 1), f32)
    deg_c = mmT(A, ones_c)
    dinv_c = jnp.where(deg_c > 0, jax.lax.rsqrt(deg_c), 0.0)

    Asc = A * dinv_r * dinv_c
    h = mmT(Asc, mmT(Asc, X))

    Xg = jnp.maximum(mm(h, Wg_ref[...]) + bg_ref[...], 0.0)
    Xg = jnp.maximum(mm(Xg, Wd_ref[...]) + bd_ref[...], 0.0)
    Xp = jnp.maximum(mm(Xg, Wp1a_ref[...]) + mm(X, Wp1b_ref[...])
                     + bp1_ref[...], 0.0)
    Xp = jnp.maximum(mm(Xp, Wp2_ref[...]) + bp2_ref[...], 0.0)
    out_ref[...] = (mm(Xp, Wpi_ref[...]) + bpi_ref[...]) * mask_ref[...]


def kernel(features, adjacency, mask, W1, b1, W2, b2, Wg, bg, Wd, bd,
           Wp1, bp1, Wp2, bp2, Wpi, bpi):
    args = (
        features, adjacency, mask.reshape(N, 1),
        W1, b1.reshape(1, -1), W2, b2.reshape(1, -1),
        Wg, bg.reshape(1, -1), Wd, bd.reshape(1, -1),
        Wp1[:256], Wp1[256:], bp1.reshape(1, -1),
        Wp2, bp2.reshape(1, -1), Wpi, bpi.reshape(1, -1),
    )
    return pl.pallas_call(
        _body,
        out_shape=jax.ShapeDtypeStruct((N, 8), jnp.float32),
    )(*args)
